# Initial kernel scaffold; baseline (speedup 1.0000x reference)
import jax
import jax.numpy as jnp
from jax.experimental import pallas as pl


def kernel(x, img_dim):
    raise NotImplementedError("write your pallas kernel here")


# speedup vs baseline: 1.8157x; 1.8157x over previous
import functools

import jax
import jax.numpy as jnp
from jax import lax
from jax.experimental import pallas as pl
from jax.experimental.pallas import tpu as pltpu

_ANCHORS_W = (10.0, 16.0, 33.0)
_ANCHORS_H = (13.0, 30.0, 23.0)
_NA = 3
_NC = 80
_C = _NC + 5


def _yolo_body(stride_ref, x_ref, o_ref, *, g):
    a = pl.program_id(1)
    stride = stride_ref[0, 0]
    p = x_ref[0, 0]
    t = p.T
    sig = jax.nn.sigmoid(t)
    e = jnp.exp(t)
    cidx = lax.broadcasted_iota(jnp.int32, t.shape, 1)
    ridx = lax.broadcasted_iota(jnp.int32, t.shape, 0)
    gx = (ridx % g).astype(jnp.float32)
    gy = (ridx // g).astype(jnp.float32)
    aw = jnp.where(a == 0, _ANCHORS_W[0], jnp.where(a == 1, _ANCHORS_W[1], _ANCHORS_W[2]))
    ah = jnp.where(a == 0, _ANCHORS_H[0], jnp.where(a == 1, _ANCHORS_H[1], _ANCHORS_H[2]))
    res = jnp.where(
        cidx == 0, (sig + gx) * stride,
        jnp.where(
            cidx == 1, (sig + gy) * stride,
            jnp.where(cidx == 2, e * aw, jnp.where(cidx == 3, e * ah, sig)),
        ),
    )
    o_ref[0, 0] = res


def kernel(x, img_dim):
    B = x.shape[0]
    g = x.shape[2]
    n = g * g
    stride = (jnp.asarray(img_dim, jnp.float32) / g).reshape(1, 1)
    xr = x.reshape(B, _NA, _C, n)
    out = pl.pallas_call(
        functools.partial(_yolo_body, g=g),
        grid=(B, _NA),
        in_specs=[
            pl.BlockSpec((1, 1), lambda b, a: (0, 0)),
            pl.BlockSpec((1, 1, _C, n), lambda b, a: (b, a, 0, 0)),
        ],
        out_specs=pl.BlockSpec((1, 1, n, _C), lambda b, a: (b, a, 0, 0)),
        out_shape=jax.ShapeDtypeStruct((B, _NA, n, _C), jnp.float32),
        compiler_params=pltpu.CompilerParams(
            dimension_semantics=("parallel", "parallel"),
        ),
    )(stride, xr)
    return out.reshape(B, _NA * n, _C)


# speedup vs baseline: 2.6761x; 1.4738x over previous
import functools

import jax
import jax.numpy as jnp
from jax import lax
from jax.experimental import pallas as pl
from jax.experimental.pallas import tpu as pltpu

_ANCHORS_W = (10.0, 16.0, 33.0)
_ANCHORS_H = (13.0, 30.0, 23.0)
_NA = 3
_NC = 80
_C = _NC + 5


def _yolo_body(stride_ref, x_ref, o_ref):
    a = pl.program_id(1)
    stride = stride_ref[0, 0]
    x3 = x_ref[0]
    g = x3.shape[1]
    sig = jax.nn.sigmoid(x3)
    e = jnp.exp(x3[2:4])
    gx = lax.broadcasted_iota(jnp.int32, (1, g, g), 2).astype(jnp.float32)
    gy = lax.broadcasted_iota(jnp.int32, (1, g, g), 1).astype(jnp.float32)
    aw = jnp.where(a == 0, _ANCHORS_W[0], jnp.where(a == 1, _ANCHORS_W[1], _ANCHORS_W[2]))
    ah = jnp.where(a == 0, _ANCHORS_H[0], jnp.where(a == 1, _ANCHORS_H[1], _ANCHORS_H[2]))
    ch0 = (sig[0:1] + gx) * stride
    ch1 = (sig[1:2] + gy) * stride
    ch2 = e[0:1] * aw
    ch3 = e[1:2] * ah
    comb = jnp.concatenate([ch0, ch1, ch2, ch3, sig[4:]], axis=0)
    o_ref[0] = comb.transpose(1, 2, 0).reshape(g * g, _C)


def kernel(x, img_dim):
    B = x.shape[0]
    g = x.shape[2]
    n = g * g
    stride = (jnp.asarray(img_dim, jnp.float32) / g).reshape(1, 1)
    out = pl.pallas_call(
        _yolo_body,
        grid=(B, _NA),
        in_specs=[
            pl.BlockSpec((1, 1), lambda b, a: (0, 0)),
            pl.BlockSpec((1, _C, g, g), lambda b, a: (b, a, 0, 0)),
        ],
        out_specs=pl.BlockSpec((1, n, _C), lambda b, a: (b, a, 0)),
        out_shape=jax.ShapeDtypeStruct((B, _NA * n, _C), jnp.float32),
        compiler_params=pltpu.CompilerParams(
            dimension_semantics=("parallel", "parallel"),
        ),
    )(stride, x)
    return out
